# baseline (device time: 266436 ns/iter reference)
import jax
import jax.numpy as jnp
from jax import lax
from jax.experimental import pallas as pl
from jax.experimental.pallas import tpu as pltpu

N_DEV = 4


def _ring_allreduce(partial, collective_id):
    R, C = partial.shape
    rows = R // N_DEV
    n_hops = 2 * (N_DEV - 1)

    def body(in_ref, out_ref, send_buf, recv_buf, send_sems, recv_sems):
        my = lax.axis_index("i")
        left = lax.rem(my + N_DEV - 1, N_DEV)
        right = lax.rem(my + 1, N_DEV)

        barrier_sem = pltpu.get_barrier_semaphore()
        for nbr in (left, right):
            pl.semaphore_signal(
                barrier_sem, inc=1,
                device_id=(nbr,), device_id_type=pl.DeviceIdType.MESH,
            )
        pl.semaphore_wait(barrier_sem, 2)

        def hop(h):
            rdma = pltpu.make_async_remote_copy(
                src_ref=send_buf.at[h],
                dst_ref=recv_buf.at[h],
                send_sem=send_sems.at[h],
                recv_sem=recv_sems.at[h],
                device_id=(right,),
                device_id_type=pl.DeviceIdType.MESH,
            )
            rdma.start()
            rdma.wait()

        send_buf[0, :, :] = in_ref[pl.ds(my * rows, rows), :]
        for h in range(N_DEV - 1):
            hop(h)
            c_recv = lax.rem(my + N_DEV - h - 1, N_DEV)
            acc = recv_buf[h] + in_ref[pl.ds(c_recv * rows, rows), :]
            send_buf[h + 1, :, :] = acc
            if h == N_DEV - 2:
                out_ref[pl.ds(c_recv * rows, rows), :] = acc

        for g in range(N_DEV - 1):
            h = N_DEV - 1 + g
            hop(h)
            c_recv = lax.rem(my + N_DEV - g, N_DEV)
            out_ref[pl.ds(c_recv * rows, rows), :] = recv_buf[h]
            if g < N_DEV - 2:
                send_buf[h + 1, :, :] = recv_buf[h]

    return pl.pallas_call(
        body,
        out_shape=jax.ShapeDtypeStruct((R, C), partial.dtype),
        in_specs=[pl.BlockSpec(memory_space=pltpu.VMEM)],
        out_specs=pl.BlockSpec(memory_space=pltpu.VMEM),
        scratch_shapes=[
            pltpu.VMEM((n_hops, rows, C), partial.dtype),
            pltpu.VMEM((n_hops, rows, C), partial.dtype),
            pltpu.SemaphoreType.DMA((n_hops,)),
            pltpu.SemaphoreType.DMA((n_hops,)),
        ],
        compiler_params=pltpu.CompilerParams(collective_id=collective_id),
    )(partial)


def kernel(x, Wq, Wk, Wv, Wo, t_emb, W_mod, W_ff1, W_ff2):
    B, S, D = x.shape
    Dh = 128
    Hl = Wq.shape[1] // Dh
    f32 = jnp.float32
    bf16 = jnp.bfloat16
    eps = 1e-5

    x = x.astype(f32)
    mod = t_emb.astype(f32) @ W_mod.astype(f32)
    sa, sha, ga, sm, shm, gm = jnp.split(mod, 6, axis=-1)

    def ln(h):
        m = jnp.mean(h, axis=-1, keepdims=True)
        v = jnp.var(h, axis=-1, keepdims=True)
        return (h - m) * lax.rsqrt(v + eps)

    xa = (ln(x) * (1.0 + sa[:, None, :]) + sha[:, None, :]).astype(bf16)
    Q = jnp.einsum("bsd,df->bsf", xa, Wq.astype(bf16),
                   preferred_element_type=f32).reshape(B, S, Hl, Dh)
    K = jnp.einsum("bsd,df->bsf", xa, Wk.astype(bf16),
                   preferred_element_type=f32).reshape(B, S, Hl, Dh)
    V = jnp.einsum("bsd,df->bsf", xa, Wv.astype(bf16),
                   preferred_element_type=f32).reshape(B, S, Hl, Dh)
    scores = jnp.einsum("bihd,bjhd->bhij", Q.astype(bf16), K.astype(bf16),
                        preferred_element_type=f32) * 0.08838834764831843
    p = jax.nn.softmax(scores, axis=-1)
    attn = jnp.einsum("bhij,bjhd->bihd", p.astype(bf16), V.astype(bf16),
                      preferred_element_type=f32)
    attn = attn.reshape(B, S, Hl * Dh)
    partial_attn = jnp.einsum("bsf,fd->bsd", attn.astype(bf16), Wo.astype(bf16),
                              preferred_element_type=f32)

    attn_full = _ring_allreduce(
        partial_attn.astype(bf16).reshape(B * S, D), collective_id=0
    ).reshape(B, S, D).astype(f32)
    x1 = x + ga[:, None, :] * attn_full

    xm = (ln(x1) * (1.0 + sm[:, None, :]) + shm[:, None, :]).astype(bf16)
    h = jnp.einsum("bsd,df->bsf", xm, W_ff1.astype(bf16),
                   preferred_element_type=f32)
    h = h * jax.nn.sigmoid(h)
    partial_ff = jnp.einsum("bsf,fd->bsd", h.astype(bf16), W_ff2.astype(bf16),
                            preferred_element_type=f32)

    ff_full = _ring_allreduce(
        partial_ff.astype(bf16).reshape(B * S, D), collective_id=1
    ).reshape(B, S, D).astype(f32)
    return x1 + gm[:, None, :] * ff_full


# device time: 199304 ns/iter; 1.3368x vs baseline; 1.3368x over previous
import jax
import jax.numpy as jnp
from jax import lax
from jax.experimental import pallas as pl
from jax.experimental.pallas import tpu as pltpu

N_DEV = 4


def _ring_allreduce(partial, collective_id):
    R, C = partial.shape
    rows = R // (2 * N_DEV)
    half = R // 2
    n_hops = 2 * (N_DEV - 1)

    def body(in_ref, out_ref,
             send_a, recv_a, send_b, recv_b,
             ss_a, rs_a, ss_b, rs_b):
        my = lax.axis_index("i")
        left = lax.rem(my + N_DEV - 1, N_DEV)
        right = lax.rem(my + 1, N_DEV)

        barrier_sem = pltpu.get_barrier_semaphore()
        for nbr in (left, right):
            pl.semaphore_signal(
                barrier_sem, inc=1,
                device_id=(nbr,), device_id_type=pl.DeviceIdType.MESH,
            )
        pl.semaphore_wait(barrier_sem, 2)

        def mk(h, sbuf, rbuf, ssem, rsem, dst):
            return pltpu.make_async_remote_copy(
                src_ref=sbuf.at[h], dst_ref=rbuf.at[h],
                send_sem=ssem.at[h], recv_sem=rsem.at[h],
                device_id=(dst,), device_id_type=pl.DeviceIdType.MESH,
            )

        def in_chunk(c, off):
            return in_ref[pl.ds(off + c * rows, rows), :]

        send_a[0, :, :] = in_chunk(my, 0)
        send_b[0, :, :] = in_chunk(my, half)
        for h in range(n_hops):
            rd_a = mk(h, send_a, recv_a, ss_a, rs_a, right)
            rd_b = mk(h, send_b, recv_b, ss_b, rs_b, left)
            rd_a.start()
            rd_b.start()
            rd_a.wait()
            rd_b.wait()
            if h < N_DEV - 1:
                s = h
                ca = lax.rem(my + N_DEV - s - 1, N_DEV)
                cb = lax.rem(my + s + 1, N_DEV)
                acc_a = recv_a[h] + in_chunk(ca, 0)
                acc_b = recv_b[h] + in_chunk(cb, half)
                send_a[h + 1, :, :] = acc_a
                send_b[h + 1, :, :] = acc_b
                if h == N_DEV - 2:
                    out_ref[pl.ds(ca * rows, rows), :] = acc_a
                    out_ref[pl.ds(half + cb * rows, rows), :] = acc_b
            else:
                g = h - (N_DEV - 1)
                ca = lax.rem(my + N_DEV - g, N_DEV)
                cb = lax.rem(my + g, N_DEV)
                out_ref[pl.ds(ca * rows, rows), :] = recv_a[h]
                out_ref[pl.ds(half + cb * rows, rows), :] = recv_b[h]
                if g < N_DEV - 2:
                    send_a[h + 1, :, :] = recv_a[h]
                    send_b[h + 1, :, :] = recv_b[h]

    buf = lambda: pltpu.VMEM((n_hops, rows, C), partial.dtype)
    sem = lambda: pltpu.SemaphoreType.DMA((n_hops,))
    return pl.pallas_call(
        body,
        out_shape=jax.ShapeDtypeStruct((R, C), partial.dtype),
        in_specs=[pl.BlockSpec(memory_space=pltpu.VMEM)],
        out_specs=pl.BlockSpec(memory_space=pltpu.VMEM),
        scratch_shapes=[buf(), buf(), buf(), buf(), sem(), sem(), sem(), sem()],
        compiler_params=pltpu.CompilerParams(collective_id=collective_id),
    )(partial)


def kernel(x, Wq, Wk, Wv, Wo, t_emb, W_mod, W_ff1, W_ff2):
    B, S, D = x.shape
    Dh = 128
    Hl = Wq.shape[1] // Dh
    f32 = jnp.float32
    bf16 = jnp.bfloat16
    eps = 1e-5

    x = x.astype(f32)
    mod = t_emb.astype(f32) @ W_mod.astype(f32)
    sa, sha, ga, sm, shm, gm = jnp.split(mod, 6, axis=-1)

    def ln(h):
        m = jnp.mean(h, axis=-1, keepdims=True)
        v = jnp.var(h, axis=-1, keepdims=True)
        return (h - m) * lax.rsqrt(v + eps)

    xa = (ln(x) * (1.0 + sa[:, None, :]) + sha[:, None, :]).astype(bf16)
    Q = jnp.einsum("bsd,df->bsf", xa, Wq.astype(bf16),
                   preferred_element_type=f32).reshape(B, S, Hl, Dh)
    K = jnp.einsum("bsd,df->bsf", xa, Wk.astype(bf16),
                   preferred_element_type=f32).reshape(B, S, Hl, Dh)
    V = jnp.einsum("bsd,df->bsf", xa, Wv.astype(bf16),
                   preferred_element_type=f32).reshape(B, S, Hl, Dh)
    scores = jnp.einsum("bihd,bjhd->bhij", Q.astype(bf16), K.astype(bf16),
                        preferred_element_type=f32) * 0.08838834764831843
    p = jax.nn.softmax(scores, axis=-1)
    attn = jnp.einsum("bhij,bjhd->bihd", p.astype(bf16), V.astype(bf16),
                      preferred_element_type=f32)
    attn = attn.reshape(B, S, Hl * Dh)
    partial_attn = jnp.einsum("bsf,fd->bsd", attn.astype(bf16), Wo.astype(bf16),
                              preferred_element_type=f32)

    attn_full = _ring_allreduce(
        partial_attn.astype(bf16).reshape(B * S, D), collective_id=0
    ).reshape(B, S, D).astype(f32)
    x1 = x + ga[:, None, :] * attn_full

    xm = (ln(x1) * (1.0 + sm[:, None, :]) + shm[:, None, :]).astype(bf16)
    h = jnp.einsum("bsd,df->bsf", xm, W_ff1.astype(bf16),
                   preferred_element_type=f32)
    h = h * jax.nn.sigmoid(h)
    partial_ff = jnp.einsum("bsf,fd->bsd", h.astype(bf16), W_ff2.astype(bf16),
                            preferred_element_type=f32)

    ff_full = _ring_allreduce(
        partial_ff.astype(bf16).reshape(B * S, D), collective_id=1
    ).reshape(B, S, D).astype(f32)
    return x1 + gm[:, None, :] * ff_full


# device time: 193255 ns/iter; 1.3787x vs baseline; 1.0313x over previous
import jax
import jax.numpy as jnp
from jax import lax
from jax.experimental import pallas as pl
from jax.experimental.pallas import tpu as pltpu

N_DEV = 4
F32 = jnp.float32
BF16 = jnp.bfloat16


def _ring_fused_allreduce(produce, epilogue, bufs):
    (send_a, recv_a, send_b, recv_b, ss_a, rs_a, ss_b, rs_b) = bufs
    my = lax.axis_index("i")
    left = lax.rem(my + N_DEV - 1, N_DEV)
    right = lax.rem(my + 1, N_DEV)
    n_hops = 2 * (N_DEV - 1)

    barrier_sem = pltpu.get_barrier_semaphore()
    for nbr in (left, right):
        pl.semaphore_signal(
            barrier_sem, inc=1,
            device_id=(nbr,), device_id_type=pl.DeviceIdType.MESH,
        )
    send_a[0, :, :] = produce(my, 0)
    send_b[0, :, :] = produce(my, 1)
    pl.semaphore_wait(barrier_sem, 2)

    def mk(h, sbuf, rbuf, ssem, rsem, dst):
        return pltpu.make_async_remote_copy(
            src_ref=sbuf.at[h], dst_ref=rbuf.at[h],
            send_sem=ssem.at[h], recv_sem=rsem.at[h],
            device_id=(dst,), device_id_type=pl.DeviceIdType.MESH,
        )

    for h in range(n_hops):
        rd_a = mk(h, send_a, recv_a, ss_a, rs_a, right)
        rd_b = mk(h, send_b, recv_b, ss_b, rs_b, left)
        rd_a.start()
        rd_b.start()
        if h < N_DEV - 1:
            ca = lax.rem(my + N_DEV - h - 1, N_DEV)
            cb = lax.rem(my + h + 1, N_DEV)
            pa = produce(ca, 0)
            pb = produce(cb, 1)
            rd_a.wait()
            rd_b.wait()
            acc_a = recv_a[h] + pa
            acc_b = recv_b[h] + pb
            send_a[h + 1, :, :] = acc_a
            send_b[h + 1, :, :] = acc_b
            if h == N_DEV - 2:
                epilogue(ca, 0, acc_a)
                epilogue(cb, 1, acc_b)
        else:
            g = h - (N_DEV - 1)
            ca = lax.rem(my + N_DEV - g, N_DEV)
            cb = lax.rem(my + g, N_DEV)
            rd_a.wait()
            rd_b.wait()
            epilogue(ca, 0, recv_a[h])
            epilogue(cb, 1, recv_b[h])
            if g < N_DEV - 2:
                send_a[h + 1, :, :] = recv_a[h]
                send_b[h + 1, :, :] = recv_b[h]


def _scratch(rows, C):
    buf = lambda: pltpu.VMEM((2 * (N_DEV - 1), rows, C), BF16)
    sem = lambda: pltpu.SemaphoreType.DMA((2 * (N_DEV - 1),))
    return [buf(), buf(), buf(), buf(), sem(), sem(), sem(), sem()]


def _attn_out_block(attn, Wo, x, ga, collective_id):
    R, C = x.shape
    rows = R // (2 * N_DEV)
    half = R // 2

    def body(attn_ref, wo_ref, x_ref, ga_ref, x1_ref, *bufs):
        def rs(c, d):
            return pl.ds(d * half + c * rows, rows)

        def produce(c, d):
            a = attn_ref[rs(c, d), :]
            return jnp.dot(a, wo_ref[:, :], preferred_element_type=F32).astype(BF16)

        def epilogue(c, d, chunk):
            r = rs(c, d)
            x1_ref[r, :] = x_ref[r, :] + ga_ref[d] * chunk.astype(F32)

        _ring_fused_allreduce(produce, epilogue, bufs)

    return pl.pallas_call(
        body,
        out_shape=jax.ShapeDtypeStruct((R, C), F32),
        in_specs=[pl.BlockSpec(memory_space=pltpu.VMEM)] * 4,
        out_specs=pl.BlockSpec(memory_space=pltpu.VMEM),
        scratch_shapes=_scratch(rows, C),
        compiler_params=pltpu.CompilerParams(collective_id=collective_id),
    )(attn, Wo, x, ga)


def _ffn_block(x1, W1, W2, sm, shm, gm, collective_id):
    R, C = x1.shape
    rows = R // (2 * N_DEV)
    half = R // 2
    eps = 1e-5

    def body(x1_ref, w1_ref, w2_ref, sm_ref, shm_ref, gm_ref, out_ref, *bufs):
        def rs(c, d):
            return pl.ds(d * half + c * rows, rows)

        def produce(c, d):
            xc = x1_ref[rs(c, d), :]
            m = jnp.mean(xc, axis=-1, keepdims=True)
            cen = xc - m
            v = jnp.mean(cen * cen, axis=-1, keepdims=True)
            xm = cen * lax.rsqrt(v + eps) * (1.0 + sm_ref[d]) + shm_ref[d]
            h = jnp.dot(xm.astype(BF16), w1_ref[:, :], preferred_element_type=F32)
            h = h * jax.nn.sigmoid(h)
            return jnp.dot(
                h.astype(BF16), w2_ref[:, :], preferred_element_type=F32
            ).astype(BF16)

        def epilogue(c, d, chunk):
            r = rs(c, d)
            out_ref[r, :] = x1_ref[r, :] + gm_ref[d] * chunk.astype(F32)

        _ring_fused_allreduce(produce, epilogue, bufs)

    return pl.pallas_call(
        body,
        out_shape=jax.ShapeDtypeStruct((R, C), F32),
        in_specs=[pl.BlockSpec(memory_space=pltpu.VMEM)] * 6,
        out_specs=pl.BlockSpec(memory_space=pltpu.VMEM),
        scratch_shapes=_scratch(rows, C),
        compiler_params=pltpu.CompilerParams(collective_id=collective_id),
    )(x1, W1, W2, sm, shm, gm)


def kernel(x, Wq, Wk, Wv, Wo, t_emb, W_mod, W_ff1, W_ff2):
    B, S, D = x.shape
    Dh = 128
    Hl = Wq.shape[1] // Dh
    eps = 1e-5

    x = x.astype(F32)
    mod = t_emb.astype(F32) @ W_mod.astype(F32)
    sa, sha, ga, sm, shm, gm = jnp.split(mod, 6, axis=-1)

    def ln(h):
        m = jnp.mean(h, axis=-1, keepdims=True)
        v = jnp.var(h, axis=-1, keepdims=True)
        return (h - m) * lax.rsqrt(v + eps)

    xa = (ln(x) * (1.0 + sa[:, None, :]) + sha[:, None, :]).astype(BF16)
    Q = jnp.einsum("bsd,df->bsf", xa, Wq.astype(BF16),
                   preferred_element_type=F32).reshape(B, S, Hl, Dh)
    K = jnp.einsum("bsd,df->bsf", xa, Wk.astype(BF16),
                   preferred_element_type=F32).reshape(B, S, Hl, Dh)
    V = jnp.einsum("bsd,df->bsf", xa, Wv.astype(BF16),
                   preferred_element_type=F32).reshape(B, S, Hl, Dh)
    scores = jnp.einsum("bihd,bjhd->bhij", Q.astype(BF16), K.astype(BF16),
                        preferred_element_type=F32) * 0.08838834764831843
    p = jax.nn.softmax(scores, axis=-1)
    attn = jnp.einsum("bhij,bjhd->bihd", p.astype(BF16), V.astype(BF16),
                      preferred_element_type=F32)
    attn = attn.reshape(B * S, Hl * Dh).astype(BF16)

    x1 = _attn_out_block(attn, Wo.astype(BF16), x.reshape(B * S, D),
                         ga, collective_id=0)

    out = _ffn_block(x1, W_ff1.astype(BF16), W_ff2.astype(BF16),
                     sm, shm, gm, collective_id=1)
    return out.reshape(B, S, D)


# device time: 165081 ns/iter; 1.6140x vs baseline; 1.1707x over previous
import jax
import jax.numpy as jnp
from jax import lax
from jax.experimental import pallas as pl
from jax.experimental.pallas import tpu as pltpu

N_DEV = 4
F32 = jnp.float32
BF16 = jnp.bfloat16


def _ring_fused_allreduce(produce, epilogue, bufs):
    (send_a, recv_a, send_b, recv_b, ss_a, rs_a, ss_b, rs_b) = bufs
    my = lax.axis_index("i")
    left = lax.rem(my + N_DEV - 1, N_DEV)
    right = lax.rem(my + 1, N_DEV)
    n_hops = 2 * (N_DEV - 1)

    barrier_sem = pltpu.get_barrier_semaphore()
    for nbr in (left, right):
        pl.semaphore_signal(
            barrier_sem, inc=1,
            device_id=(nbr,), device_id_type=pl.DeviceIdType.MESH,
        )
    send_a[0, :, :] = produce(my, 0)
    send_b[0, :, :] = produce(my, 1)
    pl.semaphore_wait(barrier_sem, 2)

    def mk(h, sbuf, rbuf, ssem, rsem, dst):
        return pltpu.make_async_remote_copy(
            src_ref=sbuf.at[h], dst_ref=rbuf.at[h],
            send_sem=ssem.at[h], recv_sem=rsem.at[h],
            device_id=(dst,), device_id_type=pl.DeviceIdType.MESH,
        )

    for h in range(n_hops):
        rd_a = mk(h, send_a, recv_a, ss_a, rs_a, right)
        rd_b = mk(h, send_b, recv_b, ss_b, rs_b, left)
        rd_a.start()
        rd_b.start()
        if h < N_DEV - 1:
            ca = lax.rem(my + N_DEV - h - 1, N_DEV)
            cb = lax.rem(my + h + 1, N_DEV)
            pa = produce(ca, 0)
            pb = produce(cb, 1)
            rd_a.wait()
            rd_b.wait()
            acc_a = recv_a[h] + pa
            acc_b = recv_b[h] + pb
            send_a[h + 1, :, :] = acc_a
            send_b[h + 1, :, :] = acc_b
            if h == N_DEV - 2:
                epilogue(ca, 0, acc_a)
                epilogue(cb, 1, acc_b)
        else:
            g = h - (N_DEV - 1)
            ca = lax.rem(my + N_DEV - g, N_DEV)
            cb = lax.rem(my + g, N_DEV)
            rd_a.wait()
            rd_b.wait()
            epilogue(ca, 0, recv_a[h])
            epilogue(cb, 1, recv_b[h])
            if g < N_DEV - 2:
                send_a[h + 1, :, :] = recv_a[h]
                send_b[h + 1, :, :] = recv_b[h]


def _scratch(rows, C):
    buf = lambda: pltpu.VMEM((2 * (N_DEV - 1), rows, C), BF16)
    sem = lambda: pltpu.SemaphoreType.DMA((2 * (N_DEV - 1),))
    return [buf(), buf(), buf(), buf(), sem(), sem(), sem(), sem()]


def _attn_out_block(Q, K, V, Wo, x, ga, n_heads, collective_id):
    R, C = x.shape
    rows = R // (2 * N_DEV)
    half = R // 2
    Dh = 128
    scale = 0.08838834764831843

    def body(q_ref, k_ref, v_ref, wo_ref, x_ref, ga_ref, x1_ref, att_buf,
             *bufs):
        def rs(c, d):
            return pl.ds(d * half + c * rows, rows)

        def produce(c, d):
            r = rs(c, d)
            kv = pl.ds(d * half, half)
            for hh in range(n_heads):
                hs = slice(hh * Dh, (hh + 1) * Dh)
                q = q_ref[r, hs]
                k = k_ref[kv, hs]
                s = lax.dot_general(
                    q, k, (((1,), (1,)), ((), ())), preferred_element_type=F32
                ) * scale
                mx = jnp.max(s, axis=-1, keepdims=True)
                e = jnp.exp(s - mx)
                p = e / jnp.sum(e, axis=-1, keepdims=True)
                o = jnp.dot(p.astype(BF16), v_ref[kv, hs],
                            preferred_element_type=F32)
                att_buf[:, hs] = o.astype(BF16)
            return jnp.dot(att_buf[:, :], wo_ref[:, :],
                           preferred_element_type=F32).astype(BF16)

        def epilogue(c, d, chunk):
            r = rs(c, d)
            x1_ref[r, :] = x_ref[r, :] + ga_ref[d] * chunk.astype(F32)

        _ring_fused_allreduce(produce, epilogue, bufs)

    return pl.pallas_call(
        body,
        out_shape=jax.ShapeDtypeStruct((R, C), F32),
        in_specs=[pl.BlockSpec(memory_space=pltpu.VMEM)] * 6,
        out_specs=pl.BlockSpec(memory_space=pltpu.VMEM),
        scratch_shapes=[pltpu.VMEM((rows, n_heads * Dh), BF16)]
        + _scratch(rows, C),
        compiler_params=pltpu.CompilerParams(collective_id=collective_id),
    )(Q, K, V, Wo, x, ga)


def _ffn_block(x1, W1, W2, sm, shm, gm, collective_id):
    R, C = x1.shape
    rows = R // (2 * N_DEV)
    half = R // 2
    eps = 1e-5

    def body(x1_ref, w1_ref, w2_ref, sm_ref, shm_ref, gm_ref, out_ref, *bufs):
        def rs(c, d):
            return pl.ds(d * half + c * rows, rows)

        def produce(c, d):
            xc = x1_ref[rs(c, d), :]
            m = jnp.mean(xc, axis=-1, keepdims=True)
            cen = xc - m
            v = jnp.mean(cen * cen, axis=-1, keepdims=True)
            xm = cen * lax.rsqrt(v + eps) * (1.0 + sm_ref[d]) + shm_ref[d]
            h = jnp.dot(xm.astype(BF16), w1_ref[:, :], preferred_element_type=F32)
            h = h * jax.nn.sigmoid(h)
            return jnp.dot(
                h.astype(BF16), w2_ref[:, :], preferred_element_type=F32
            ).astype(BF16)

        def epilogue(c, d, chunk):
            r = rs(c, d)
            out_ref[r, :] = x1_ref[r, :] + gm_ref[d] * chunk.astype(F32)

        _ring_fused_allreduce(produce, epilogue, bufs)

    return pl.pallas_call(
        body,
        out_shape=jax.ShapeDtypeStruct((R, C), F32),
        in_specs=[pl.BlockSpec(memory_space=pltpu.VMEM)] * 6,
        out_specs=pl.BlockSpec(memory_space=pltpu.VMEM),
        scratch_shapes=_scratch(rows, C),
        compiler_params=pltpu.CompilerParams(collective_id=collective_id),
    )(x1, W1, W2, sm, shm, gm)


def kernel(x, Wq, Wk, Wv, Wo, t_emb, W_mod, W_ff1, W_ff2):
    B, S, D = x.shape
    Dh = 128
    Hl = Wq.shape[1] // Dh
    eps = 1e-5

    x = x.astype(F32)
    mod = t_emb.astype(F32) @ W_mod.astype(F32)
    sa, sha, ga, sm, shm, gm = jnp.split(mod, 6, axis=-1)

    def ln(h):
        m = jnp.mean(h, axis=-1, keepdims=True)
        v = jnp.var(h, axis=-1, keepdims=True)
        return (h - m) * lax.rsqrt(v + eps)

    xa = (ln(x) * (1.0 + sa[:, None, :]) + sha[:, None, :]).astype(BF16)
    xa = xa.reshape(B * S, D)
    Q = (xa @ Wq.astype(BF16)).astype(BF16)
    K = (xa @ Wk.astype(BF16)).astype(BF16)
    V = (xa @ Wv.astype(BF16)).astype(BF16)

    x1 = _attn_out_block(Q, K, V, Wo.astype(BF16), x.reshape(B * S, D),
                         ga, Hl, collective_id=0)

    out = _ffn_block(x1, W_ff1.astype(BF16), W_ff2.astype(BF16),
                     sm, shm, gm, collective_id=1)
    return out.reshape(B, S, D)
